# baseline (speedup 1.0000x reference)
import functools

import jax
import jax.numpy as jnp
from jax.experimental import pallas as pl


def _top2_into(lg, iota, ow_ref, oi_ref, l, lo, hi, num_experts):
    neg_inf = jnp.float32(-jnp.inf)
    l1 = jnp.max(lg, axis=0)
    i1 = jnp.min(jnp.where(lg == l1[None, :], iota, num_experts), axis=0)
    masked = jnp.where(iota == i1[None, :], neg_inf, lg)
    l2 = jnp.max(masked, axis=0)
    i2 = jnp.min(jnp.where(masked == l2[None, :], iota, num_experts), axis=0)
    r = jnp.exp(l2 - l1)
    w1 = 1.0 / (1.0 + r)
    ow_ref[l, 0, lo:hi] = w1
    ow_ref[l, 1, lo:hi] = 1.0 - w1
    oi_ref[l, 0, lo:hi] = i1.astype(jnp.int32)
    oi_ref[l, 1, lo:hi] = i2.astype(jnp.int32)


def _router_kernel(xa_ref, xb_ref, w_ref, ow_ref, oi_ref, *, num_layers,
                   num_experts):
    w = w_ref[...]
    half = xa_ref.shape[0]
    iota = jax.lax.broadcasted_iota(jnp.int32, (num_experts, half), 0)
    for part, x_ref in enumerate((xa_ref, xb_ref)):
        x = x_ref[...]
        logits = jax.lax.dot_general(
            w, x,
            dimension_numbers=(((1,), (1,)), ((), ())),
            preferred_element_type=jnp.float32,
        )
        lo, hi = part * half, (part + 1) * half
        for l in range(num_layers):
            lg = logits[l * num_experts:(l + 1) * num_experts, :]
            _top2_into(lg, iota, ow_ref, oi_ref, l, lo, hi, num_experts)


@jax.jit
def kernel(hidden_states, router_weights):
    t, h = hidden_states.shape
    num_layers, num_experts, _ = router_weights.shape
    w2d = router_weights.reshape(num_layers * num_experts, h)
    half = 512
    tile = 2 * half
    grid = (t // tile,)
    kfn = functools.partial(_router_kernel, num_layers=num_layers,
                            num_experts=num_experts)
    ow, oi = pl.pallas_call(
        kfn,
        grid=grid,
        in_specs=[
            pl.BlockSpec((half, h), lambda i: (2 * i, 0)),
            pl.BlockSpec((half, h), lambda i: (2 * i + 1, 0)),
            pl.BlockSpec((num_layers * num_experts, h), lambda i: (0, 0)),
        ],
        out_specs=[
            pl.BlockSpec((num_layers, 2, tile), lambda i: (0, 0, i)),
            pl.BlockSpec((num_layers, 2, tile), lambda i: (0, 0, i)),
        ],
        out_shape=[
            jax.ShapeDtypeStruct((num_layers, 2, t), jnp.float32),
            jax.ShapeDtypeStruct((num_layers, 2, t), jnp.int32),
        ],
    )(hidden_states, hidden_states, w2d)
    return jnp.swapaxes(ow, 1, 2), jnp.swapaxes(oi, 1, 2)
